# Initial kernel scaffold; baseline (speedup 1.0000x reference)
import jax
import jax.numpy as jnp
from jax.experimental import pallas as pl


def kernel(xyz, W_proj, b_proj, W_sa0, b_sa0, W_sa1, b_sa1, W_sa2, b_sa2, W_d0, b_d0, W_d1, b_d1):
    raise NotImplementedError("write your pallas kernel here")


# speedup vs baseline: 5.1567x; 5.1567x over previous
import functools

import jax
import jax.numpy as jnp
from jax import lax
from jax.experimental import pallas as pl
from jax.experimental.pallas import tpu as pltpu
from jax.experimental.pallas import tpu_sc as plsc

_K = 32
_NC = 2
_NS = 16
_NW = _NC * _NS


def _knn_offsets(pts, cents, Wx, bvec, tile_s):
  B, Np, _ = pts.shape
  S = cents.shape[1]
  C = Wx.shape[1]
  centsT = cents.transpose(0, 2, 1)
  grid = (B, S // tile_s)

  def kern(c_ref, cT_ref, p_ref, wx_ref, b_ref, idx_ref, o_ref, d_ref):
    b = pl.program_id(0)
    px = p_ref[0, :, 0:1]
    py = p_ref[0, :, 1:2]
    pz = p_ref[0, :, 2:3]
    cxT = cT_ref[0, 0:1, :]
    cyT = cT_ref[0, 1:2, :]
    czT = cT_ref[0, 2:3, :]
    d_ref[...] = (px - cxT) ** 2 + (py - cyT) ** 2 + (pz - czT) ** 2

    def body(r, carry):
      d = d_ref[...]
      iota = lax.broadcasted_iota(jnp.int32, (Np, tile_s), 0)
      m = jnp.min(d, axis=0, keepdims=True)
      hit = d <= m
      row = jnp.min(jnp.where(hit, iota, Np), axis=0, keepdims=True)
      idx_ref[0, pl.ds(r, 1), :] = row + b * Np
      d_ref[...] = jnp.where(iota == row, jnp.inf, d)
      return carry

    lax.fori_loop(0, _K, body, 0)
    cx = c_ref[0, :, 0:1]
    cy = c_ref[0, :, 1:2]
    cz = c_ref[0, :, 2:3]
    o_ref[0] = b_ref[0:1, :] - (
        cx * wx_ref[0:1, :] + cy * wx_ref[1:2, :] + cz * wx_ref[2:3, :])

  return pl.pallas_call(
      kern,
      grid=grid,
      in_specs=[
          pl.BlockSpec((1, tile_s, 3), lambda b, s: (b, s, 0)),
          pl.BlockSpec((1, 3, tile_s), lambda b, s: (b, 0, s)),
          pl.BlockSpec((1, Np, 3), lambda b, s: (b, 0, 0)),
          pl.BlockSpec((3, C), lambda b, s: (0, 0)),
          pl.BlockSpec((1, C), lambda b, s: (0, 0)),
      ],
      out_specs=[
          pl.BlockSpec((1, _K, tile_s), lambda b, s: (b, 0, s)),
          pl.BlockSpec((1, tile_s, C), lambda b, s: (b, s, 0)),
      ],
      out_shape=[
          jax.ShapeDtypeStruct((B, _K, S), jnp.int32),
          jax.ShapeDtypeStruct((B, S, C), jnp.float32),
      ],
      scratch_shapes=[pltpu.VMEM((Np, tile_s), jnp.float32)],
  )(cents, centsT, pts, Wx, bvec)


def _q0(xyz, A, c0):
  B, N, _ = xyz.shape
  C = A.shape[1]

  def kern(x_ref, a_ref, c_ref, q_ref):
    x = x_ref[0, :, 0:1]
    y = x_ref[0, :, 1:2]
    z = x_ref[0, :, 2:3]
    q_ref[0] = (x * a_ref[0:1, :] + y * a_ref[1:2, :] + z * a_ref[2:3, :]
                + c_ref[0:1, :])

  return pl.pallas_call(
      kern,
      grid=(B,),
      in_specs=[
          pl.BlockSpec((1, N, 3), lambda b: (b, 0, 0)),
          pl.BlockSpec((3, C), lambda b: (0, 0)),
          pl.BlockSpec((1, C), lambda b: (0, 0)),
      ],
      out_specs=pl.BlockSpec((1, N, C), lambda b: (b, 0, 0)),
      out_shape=jax.ShapeDtypeStruct((B, N, C), jnp.float32),
  )(xyz, A, c0)


def _q_layer(feats, pts, Wf, Wx):
  B, Np, Cin = feats.shape
  C = Wf.shape[1]

  def kern(f_ref, p_ref, wf_ref, wx_ref, q_ref):
    x = p_ref[0, :, 0:1]
    y = p_ref[0, :, 1:2]
    z = p_ref[0, :, 2:3]
    q = jnp.dot(f_ref[0], wf_ref[...], preferred_element_type=jnp.float32)
    q_ref[0] = q + (x * wx_ref[0:1, :] + y * wx_ref[1:2, :]
                    + z * wx_ref[2:3, :])

  return pl.pallas_call(
      kern,
      grid=(B,),
      in_specs=[
          pl.BlockSpec((1, Np, Cin), lambda b: (b, 0, 0)),
          pl.BlockSpec((1, Np, 3), lambda b: (b, 0, 0)),
          pl.BlockSpec((Cin, C), lambda b: (0, 0)),
          pl.BlockSpec((3, C), lambda b: (0, 0)),
      ],
      out_specs=pl.BlockSpec((1, Np, C), lambda b: (b, 0, 0)),
      out_shape=jax.ShapeDtypeStruct((B, Np, C), jnp.float32),
  )(feats, pts, Wf, Wx)


def _gather_max(Qf, idxf, off, C):
  R_total = off.shape[0]
  R = R_total // _NW
  G = 4
  steps = R // G
  mesh = plsc.VectorSubcoreMesh(core_axis_name="c", subcore_axis_name="s",
                                num_cores=_NC, num_subcores=_NS)

  @functools.partial(
      pl.kernel,
      mesh=mesh,
      out_type=jax.ShapeDtypeStruct((R_total, C), jnp.float32),
      scratch_types=[
          pltpu.VMEM((G * _K,), jnp.int32),
          pltpu.VMEM((G * _K, C), jnp.float32),
          pltpu.VMEM((G, C), jnp.float32),
          pltpu.VMEM((G, C), jnp.float32),
          pltpu.SemaphoreType.DMA,
      ],
  )
  def kern(q_hbm, idx_hbm, o_hbm, out_hbm, idx_v, rows_v, o_v, out_v, sem):
    wid = lax.axis_index("s") * _NC + lax.axis_index("c")

    def step(t, carry):
      base = wid * R + t * G
      pltpu.sync_copy(idx_hbm.at[pl.ds(base * _K, G * _K)], idx_v)
      pltpu.async_copy(q_hbm.at[idx_v], rows_v, sem).wait()
      pltpu.sync_copy(o_hbm.at[pl.ds(base, G)], o_v)
      for g in range(G):
        def cb_body(cb, _, g=g):
          def j_body(j, m, g=g):
            return jnp.maximum(m, rows_v[g * _K + j, pl.ds(cb * 16, 16)])
          m0 = rows_v[g * _K, pl.ds(cb * 16, 16)]
          m = lax.fori_loop(1, _K, j_body, m0)
          res = jnp.maximum(m + o_v[g, pl.ds(cb * 16, 16)], 0.0)
          out_v[g, pl.ds(cb * 16, 16)] = res
          return 0
        lax.fori_loop(0, C // 16, cb_body, 0)
      pltpu.sync_copy(out_v, out_hbm.at[pl.ds(base, G)])
      return carry

    lax.fori_loop(0, steps, step, 0)

  return kern(Qf, idxf, off)


def _head(feats, Wd0, bd0, Wd1, bd1):
  B, S, C = feats.shape
  D0 = Wd0.shape[1]
  D1 = Wd1.shape[1]

  def kern(f_ref, w0_ref, b0_ref, w1_ref, b1_ref, out_ref):
    f = f_ref[0]
    fmax = jnp.max(f, axis=0, keepdims=True)
    fmean = jnp.mean(f, axis=0, keepdims=True)
    x = jnp.concatenate([fmax, fmean], axis=1)
    h = jnp.dot(x, w0_ref[...], preferred_element_type=jnp.float32)
    h = jnp.maximum(h + b0_ref[0:1, :], 0.0)
    y = jnp.dot(h, w1_ref[...], preferred_element_type=jnp.float32)
    out_ref[0] = y + b1_ref[0:1, :]

  out = pl.pallas_call(
      kern,
      grid=(B,),
      in_specs=[
          pl.BlockSpec((1, S, C), lambda b: (b, 0, 0)),
          pl.BlockSpec((2 * C, D0), lambda b: (0, 0)),
          pl.BlockSpec((1, D0), lambda b: (0, 0)),
          pl.BlockSpec((D0, D1), lambda b: (0, 0)),
          pl.BlockSpec((1, D1), lambda b: (0, 0)),
      ],
      out_specs=pl.BlockSpec((1, 1, D1), lambda b: (b, 0, 0)),
      out_shape=jax.ShapeDtypeStruct((B, 1, D1), jnp.float32),
  )(feats, Wd0, bd0, Wd1, bd1)
  return out.reshape(B, D1)


def kernel(xyz, W_proj, b_proj, W_sa0, b_sa0, W_sa1, b_sa1, W_sa2, b_sa2,
           W_d0, b_d0, W_d1, b_d1):
  B, N, _ = xyz.shape
  fps = (1024, 256, 64)

  Wf0, Wx0 = W_sa0[:-3], W_sa0[-3:]
  Wf1, Wx1 = W_sa1[:-3], W_sa1[-3:]
  Wf2, Wx2 = W_sa2[:-3], W_sa2[-3:]
  A0 = W_proj @ Wf0 + Wx0
  c0 = (b_proj @ Wf0)[None, :]

  pts0 = xyz
  c1 = pts0[:, :: N // fps[0]]
  c2 = c1[:, :: fps[0] // fps[1]]
  c3 = c2[:, :: fps[1] // fps[2]]

  C0 = Wf0.shape[1]
  Q0 = _q0(xyz, A0, c0)
  idx0, off0 = _knn_offsets(pts0, c1, Wx0, b_sa0[None, :], 128)
  f1 = _gather_max(Q0.reshape(B * N, C0), idx0.transpose(0, 2, 1).reshape(-1),
                   off0.reshape(B * fps[0], C0), C0)
  f1 = f1.reshape(B, fps[0], C0)

  C1 = Wf1.shape[1]
  Q1 = _q_layer(f1, c1, Wf1, Wx1)
  idx1, off1 = _knn_offsets(c1, c2, Wx1, b_sa1[None, :], 128)
  f2 = _gather_max(Q1.reshape(B * fps[0], C1), idx1.transpose(0, 2, 1).reshape(-1),
                   off1.reshape(B * fps[1], C1), C1)
  f2 = f2.reshape(B, fps[1], C1)

  C2 = Wf2.shape[1]
  Q2 = _q_layer(f2, c2, Wf2, Wx2)
  idx2, off2 = _knn_offsets(c2, c3, Wx2, b_sa2[None, :], 64)
  f3 = _gather_max(Q2.reshape(B * fps[1], C2), idx2.transpose(0, 2, 1).reshape(-1),
                   off2.reshape(B * fps[2], C2), C2)
  f3 = f3.reshape(B, fps[2], C2)

  return _head(f3, W_d0, b_d0[None, :], W_d1, b_d1[None, :])


# speedup vs baseline: 7.5537x; 1.4648x over previous
import functools

import jax
import jax.numpy as jnp
from jax import lax
from jax.experimental import pallas as pl
from jax.experimental.pallas import tpu as pltpu
from jax.experimental.pallas import tpu_sc as plsc

_K = 32
_NC = 2
_NS = 16
_NW = _NC * _NS


def _knn_offsets(pts, cents, Wx, bvec, tile_s):
  B, Np, _ = pts.shape
  S = cents.shape[1]
  C = Wx.shape[1]
  centsT = cents.transpose(0, 2, 1)
  grid = (B, S // tile_s)

  def kern(c_ref, cT_ref, p_ref, wx_ref, b_ref, idx_ref, o_ref, d_ref):
    b = pl.program_id(0)
    px = p_ref[0, :, 0:1]
    py = p_ref[0, :, 1:2]
    pz = p_ref[0, :, 2:3]
    cxT = cT_ref[0, 0:1, :]
    cyT = cT_ref[0, 1:2, :]
    czT = cT_ref[0, 2:3, :]
    d_ref[...] = (px - cxT) ** 2 + (py - cyT) ** 2 + (pz - czT) ** 2

    def body(r, carry):
      d = d_ref[...]
      iota = lax.broadcasted_iota(jnp.int32, (Np, tile_s), 0)
      m = jnp.min(d, axis=0, keepdims=True)
      hit = d <= m
      row = jnp.min(jnp.where(hit, iota, Np), axis=0, keepdims=True)
      idx_ref[0, pl.ds(r, 1), :] = row + b * Np
      d_ref[...] = jnp.where(iota == row, jnp.inf, d)
      return carry

    lax.fori_loop(0, _K, body, 0)
    cx = c_ref[0, :, 0:1]
    cy = c_ref[0, :, 1:2]
    cz = c_ref[0, :, 2:3]
    o_ref[0] = b_ref[0:1, :] - (
        cx * wx_ref[0:1, :] + cy * wx_ref[1:2, :] + cz * wx_ref[2:3, :])

  return pl.pallas_call(
      kern,
      grid=grid,
      in_specs=[
          pl.BlockSpec((1, tile_s, 3), lambda b, s: (b, s, 0)),
          pl.BlockSpec((1, 3, tile_s), lambda b, s: (b, 0, s)),
          pl.BlockSpec((1, Np, 3), lambda b, s: (b, 0, 0)),
          pl.BlockSpec((3, C), lambda b, s: (0, 0)),
          pl.BlockSpec((1, C), lambda b, s: (0, 0)),
      ],
      out_specs=[
          pl.BlockSpec((1, _K, tile_s), lambda b, s: (b, 0, s)),
          pl.BlockSpec((1, tile_s, C), lambda b, s: (b, s, 0)),
      ],
      out_shape=[
          jax.ShapeDtypeStruct((B, _K, S), jnp.int32),
          jax.ShapeDtypeStruct((B, S, C), jnp.float32),
      ],
      scratch_shapes=[pltpu.VMEM((Np, tile_s), jnp.float32)],
  )(cents, centsT, pts, Wx, bvec)


def _q0(xyz, A, c0):
  B, N, _ = xyz.shape
  C = A.shape[1]

  def kern(x_ref, a_ref, c_ref, q_ref):
    x = x_ref[0, :, 0:1]
    y = x_ref[0, :, 1:2]
    z = x_ref[0, :, 2:3]
    q_ref[0] = (x * a_ref[0:1, :] + y * a_ref[1:2, :] + z * a_ref[2:3, :]
                + c_ref[0:1, :])

  return pl.pallas_call(
      kern,
      grid=(B,),
      in_specs=[
          pl.BlockSpec((1, N, 3), lambda b: (b, 0, 0)),
          pl.BlockSpec((3, C), lambda b: (0, 0)),
          pl.BlockSpec((1, C), lambda b: (0, 0)),
      ],
      out_specs=pl.BlockSpec((1, N, C), lambda b: (b, 0, 0)),
      out_shape=jax.ShapeDtypeStruct((B, N, C), jnp.float32),
  )(xyz, A, c0)


def _q_layer(feats, pts, Wf, Wx):
  B, Np, Cin = feats.shape
  C = Wf.shape[1]

  def kern(f_ref, p_ref, wf_ref, wx_ref, q_ref):
    x = p_ref[0, :, 0:1]
    y = p_ref[0, :, 1:2]
    z = p_ref[0, :, 2:3]
    q = jnp.dot(f_ref[0], wf_ref[...], preferred_element_type=jnp.float32)
    q_ref[0] = q + (x * wx_ref[0:1, :] + y * wx_ref[1:2, :]
                    + z * wx_ref[2:3, :])

  return pl.pallas_call(
      kern,
      grid=(B,),
      in_specs=[
          pl.BlockSpec((1, Np, Cin), lambda b: (b, 0, 0)),
          pl.BlockSpec((1, Np, 3), lambda b: (b, 0, 0)),
          pl.BlockSpec((Cin, C), lambda b: (0, 0)),
          pl.BlockSpec((3, C), lambda b: (0, 0)),
      ],
      out_specs=pl.BlockSpec((1, Np, C), lambda b: (b, 0, 0)),
      out_shape=jax.ShapeDtypeStruct((B, Np, C), jnp.float32),
  )(feats, pts, Wf, Wx)


def _gather_max(Qf, idxf, off, C):
  R_total = off.shape[0]
  R = R_total // _NW
  G = 4 if C <= 256 else 2
  steps = R // G
  mesh = plsc.VectorSubcoreMesh(core_axis_name="c", subcore_axis_name="s",
                                num_cores=_NC, num_subcores=_NS)

  assert steps % 2 == 0

  @functools.partial(
      pl.kernel,
      mesh=mesh,
      out_type=jax.ShapeDtypeStruct((R_total, C), jnp.float32),
      scratch_types=[
          [pltpu.VMEM((G * _K,), jnp.int32) for _ in range(2)],
          [pltpu.VMEM((G * _K, C), jnp.float32) for _ in range(2)],
          [pltpu.VMEM((G, C), jnp.float32) for _ in range(2)],
          pltpu.VMEM((G, C), jnp.float32),
          [pltpu.SemaphoreType.DMA for _ in range(2)],
      ],
  )
  def kern(q_hbm, idx_hbm, o_hbm, out_hbm, idx_v, rows_v, o_v, out_v, sem):
    wid = lax.axis_index("s") * _NC + lax.axis_index("c")

    def start(t, p):
      base = wid * R + t * G
      pltpu.sync_copy(idx_hbm.at[pl.ds(base * _K, G * _K)], idx_v[p])
      pltpu.async_copy(q_hbm.at[idx_v[p]], rows_v[p], sem[p])
      pltpu.sync_copy(o_hbm.at[pl.ds(base, G)], o_v[p])

    def finish(t, p):
      pltpu.make_async_copy(q_hbm.at[idx_v[p]], rows_v[p], sem[p]).wait()
      base = wid * R + t * G
      for g in range(G):
        def cb_body(cb, _, g=g):
          def j_body(j, m, g=g):
            return jnp.maximum(m, rows_v[p][g * _K + j, pl.ds(cb * 16, 16)])
          m0 = rows_v[p][g * _K, pl.ds(cb * 16, 16)]
          m = lax.fori_loop(1, _K, j_body, m0)
          res = jnp.maximum(m + o_v[p][g, pl.ds(cb * 16, 16)], 0.0)
          out_v[g, pl.ds(cb * 16, 16)] = res
          return 0
        lax.fori_loop(0, C // 16, cb_body, 0)
      pltpu.sync_copy(out_v, out_hbm.at[pl.ds(base, G)])

    start(0, 0)

    def pair(t2, carry):
      t = 2 * t2
      start(t + 1, 1)
      finish(t, 0)

      @pl.when(t2 < steps // 2 - 1)
      def _():
        start(t + 2, 0)

      finish(t + 1, 1)
      return carry

    lax.fori_loop(0, steps // 2, pair, 0)

  return kern(Qf, idxf, off)


def _head(feats, Wd0, bd0, Wd1, bd1):
  B, S, C = feats.shape
  D0 = Wd0.shape[1]
  D1 = Wd1.shape[1]

  def kern(f_ref, w0_ref, b0_ref, w1_ref, b1_ref, out_ref):
    f = f_ref[0]
    fmax = jnp.max(f, axis=0, keepdims=True)
    fmean = jnp.mean(f, axis=0, keepdims=True)
    x = jnp.concatenate([fmax, fmean], axis=1)
    h = jnp.dot(x, w0_ref[...], preferred_element_type=jnp.float32)
    h = jnp.maximum(h + b0_ref[0:1, :], 0.0)
    y = jnp.dot(h, w1_ref[...], preferred_element_type=jnp.float32)
    out_ref[0] = y + b1_ref[0:1, :]

  out = pl.pallas_call(
      kern,
      grid=(B,),
      in_specs=[
          pl.BlockSpec((1, S, C), lambda b: (b, 0, 0)),
          pl.BlockSpec((2 * C, D0), lambda b: (0, 0)),
          pl.BlockSpec((1, D0), lambda b: (0, 0)),
          pl.BlockSpec((D0, D1), lambda b: (0, 0)),
          pl.BlockSpec((1, D1), lambda b: (0, 0)),
      ],
      out_specs=pl.BlockSpec((1, 1, D1), lambda b: (b, 0, 0)),
      out_shape=jax.ShapeDtypeStruct((B, 1, D1), jnp.float32),
  )(feats, Wd0, bd0, Wd1, bd1)
  return out.reshape(B, D1)


def kernel(xyz, W_proj, b_proj, W_sa0, b_sa0, W_sa1, b_sa1, W_sa2, b_sa2,
           W_d0, b_d0, W_d1, b_d1):
  B, N, _ = xyz.shape
  fps = (1024, 256, 64)

  Wf0, Wx0 = W_sa0[:-3], W_sa0[-3:]
  Wf1, Wx1 = W_sa1[:-3], W_sa1[-3:]
  Wf2, Wx2 = W_sa2[:-3], W_sa2[-3:]
  A0 = W_proj @ Wf0 + Wx0
  c0 = (b_proj @ Wf0)[None, :]

  pts0 = xyz
  c1 = pts0[:, :: N // fps[0]]
  c2 = c1[:, :: fps[0] // fps[1]]
  c3 = c2[:, :: fps[1] // fps[2]]

  C0 = Wf0.shape[1]
  Q0 = _q0(xyz, A0, c0)
  idx0, off0 = _knn_offsets(pts0, c1, Wx0, b_sa0[None, :], 256)
  f1 = _gather_max(Q0.reshape(B * N, C0), idx0.transpose(0, 2, 1).reshape(-1),
                   off0.reshape(B * fps[0], C0), C0)
  f1 = f1.reshape(B, fps[0], C0)

  C1 = Wf1.shape[1]
  Q1 = _q_layer(f1, c1, Wf1, Wx1)
  idx1, off1 = _knn_offsets(c1, c2, Wx1, b_sa1[None, :], 128)
  f2 = _gather_max(Q1.reshape(B * fps[0], C1), idx1.transpose(0, 2, 1).reshape(-1),
                   off1.reshape(B * fps[1], C1), C1)
  f2 = f2.reshape(B, fps[1], C1)

  C2 = Wf2.shape[1]
  Q2 = _q_layer(f2, c2, Wf2, Wx2)
  idx2, off2 = _knn_offsets(c2, c3, Wx2, b_sa2[None, :], 64)
  f3 = _gather_max(Q2.reshape(B * fps[1], C2), idx2.transpose(0, 2, 1).reshape(-1),
                   off2.reshape(B * fps[2], C2), C2)
  f3 = f3.reshape(B, fps[2], C2)

  return _head(f3, W_d0, b_d0[None, :], W_d1, b_d1[None, :])


# speedup vs baseline: 7.5572x; 1.0005x over previous
import functools

import jax
import jax.numpy as jnp
from jax import lax
from jax.experimental import pallas as pl
from jax.experimental.pallas import tpu as pltpu
from jax.experimental.pallas import tpu_sc as plsc

_K = 32
_NC = 2
_NS = 16
_NW = _NC * _NS


def _knn_offsets(pts, cents, Wx, bvec, tile_s):
  B, Np, _ = pts.shape
  S = cents.shape[1]
  C = Wx.shape[1]
  centsT = cents.transpose(0, 2, 1)
  grid = (B, S // tile_s)

  def kern(c_ref, cT_ref, p_ref, wx_ref, b_ref, idx_ref, o_ref, d_ref):
    b = pl.program_id(0)
    px = p_ref[0, :, 0:1]
    py = p_ref[0, :, 1:2]
    pz = p_ref[0, :, 2:3]
    cxT = cT_ref[0, 0:1, :]
    cyT = cT_ref[0, 1:2, :]
    czT = cT_ref[0, 2:3, :]
    d_ref[...] = (px - cxT) ** 2 + (py - cyT) ** 2 + (pz - czT) ** 2

    def body(r, carry):
      d = d_ref[...]
      iota = lax.broadcasted_iota(jnp.int32, (Np, tile_s), 0)
      m = jnp.min(d, axis=0, keepdims=True)
      hit = d <= m
      row = jnp.min(jnp.where(hit, iota, Np), axis=0, keepdims=True)
      idx_ref[0, pl.ds(r, 1), :] = row + b * Np
      d_ref[...] = jnp.where(iota == row, jnp.inf, d)
      return carry

    lax.fori_loop(0, _K, body, 0)
    cx = c_ref[0, :, 0:1]
    cy = c_ref[0, :, 1:2]
    cz = c_ref[0, :, 2:3]
    o_ref[0] = b_ref[0:1, :] - (
        cx * wx_ref[0:1, :] + cy * wx_ref[1:2, :] + cz * wx_ref[2:3, :])

  return pl.pallas_call(
      kern,
      grid=grid,
      in_specs=[
          pl.BlockSpec((1, tile_s, 3), lambda b, s: (b, s, 0)),
          pl.BlockSpec((1, 3, tile_s), lambda b, s: (b, 0, s)),
          pl.BlockSpec((1, Np, 3), lambda b, s: (b, 0, 0)),
          pl.BlockSpec((3, C), lambda b, s: (0, 0)),
          pl.BlockSpec((1, C), lambda b, s: (0, 0)),
      ],
      out_specs=[
          pl.BlockSpec((1, _K, tile_s), lambda b, s: (b, 0, s)),
          pl.BlockSpec((1, tile_s, C), lambda b, s: (b, s, 0)),
      ],
      out_shape=[
          jax.ShapeDtypeStruct((B, _K, S), jnp.int32),
          jax.ShapeDtypeStruct((B, S, C), jnp.float32),
      ],
      scratch_shapes=[pltpu.VMEM((Np, tile_s), jnp.float32)],
  )(cents, centsT, pts, Wx, bvec)


def _q0(xyz, A, c0):
  B, N, _ = xyz.shape
  C = A.shape[1]

  def kern(x_ref, a_ref, c_ref, q_ref):
    x = x_ref[0, :, 0:1]
    y = x_ref[0, :, 1:2]
    z = x_ref[0, :, 2:3]
    q_ref[0] = (x * a_ref[0:1, :] + y * a_ref[1:2, :] + z * a_ref[2:3, :]
                + c_ref[0:1, :])

  return pl.pallas_call(
      kern,
      grid=(B,),
      in_specs=[
          pl.BlockSpec((1, N, 3), lambda b: (b, 0, 0)),
          pl.BlockSpec((3, C), lambda b: (0, 0)),
          pl.BlockSpec((1, C), lambda b: (0, 0)),
      ],
      out_specs=pl.BlockSpec((1, N, C), lambda b: (b, 0, 0)),
      out_shape=jax.ShapeDtypeStruct((B, N, C), jnp.float32),
  )(xyz, A, c0)


def _q_layer(feats, pts, Wf, Wx):
  B, Np, Cin = feats.shape
  C = Wf.shape[1]

  def kern(f_ref, p_ref, wf_ref, wx_ref, q_ref):
    x = p_ref[0, :, 0:1]
    y = p_ref[0, :, 1:2]
    z = p_ref[0, :, 2:3]
    q = jnp.dot(f_ref[0], wf_ref[...], preferred_element_type=jnp.float32)
    q_ref[0] = q + (x * wx_ref[0:1, :] + y * wx_ref[1:2, :]
                    + z * wx_ref[2:3, :])

  return pl.pallas_call(
      kern,
      grid=(B,),
      in_specs=[
          pl.BlockSpec((1, Np, Cin), lambda b: (b, 0, 0)),
          pl.BlockSpec((1, Np, 3), lambda b: (b, 0, 0)),
          pl.BlockSpec((Cin, C), lambda b: (0, 0)),
          pl.BlockSpec((3, C), lambda b: (0, 0)),
      ],
      out_specs=pl.BlockSpec((1, Np, C), lambda b: (b, 0, 0)),
      out_shape=jax.ShapeDtypeStruct((B, Np, C), jnp.float32),
  )(feats, pts, Wf, Wx)


def _gather_max(Qf, idxf, off, C):
  R_total = off.shape[0]
  R = R_total // _NW
  G = 4 if C <= 256 else 2
  steps = R // G
  mesh = plsc.VectorSubcoreMesh(core_axis_name="c", subcore_axis_name="s",
                                num_cores=_NC, num_subcores=_NS)

  assert steps % 2 == 0

  @functools.partial(
      pl.kernel,
      mesh=mesh,
      out_type=jax.ShapeDtypeStruct((R_total, C), jnp.float32),
      scratch_types=[
          [pltpu.VMEM((G * _K,), jnp.int32) for _ in range(2)],
          [pltpu.VMEM((G * _K, C), jnp.float32) for _ in range(2)],
          [pltpu.VMEM((G, C), jnp.float32) for _ in range(2)],
          pltpu.VMEM((G, C), jnp.float32),
          [pltpu.SemaphoreType.DMA for _ in range(2)],
      ],
  )
  def kern(q_hbm, idx_hbm, o_hbm, out_hbm, idx_v, rows_v, o_v, out_v, sem):
    wid = lax.axis_index("s") * _NC + lax.axis_index("c")

    def start(t, p):
      base = wid * R + t * G
      pltpu.sync_copy(idx_hbm.at[pl.ds(base * _K, G * _K)], idx_v[p])
      pltpu.async_copy(q_hbm.at[idx_v[p]], rows_v[p], sem[p])
      pltpu.sync_copy(o_hbm.at[pl.ds(base, G)], o_v[p])

    def finish(t, p):
      pltpu.make_async_copy(q_hbm.at[idx_v[p]], rows_v[p], sem[p]).wait()
      base = wid * R + t * G
      for g in range(G):
        def cb_body(cb, _, g=g):
          def j_body(j, m, g=g):
            return jnp.maximum(m, rows_v[p][g * _K + j, pl.ds(cb * 16, 16)])
          m0 = rows_v[p][g * _K, pl.ds(cb * 16, 16)]
          m = lax.fori_loop(1, _K, j_body, m0)
          res = jnp.maximum(m + o_v[p][g, pl.ds(cb * 16, 16)], 0.0)
          out_v[g, pl.ds(cb * 16, 16)] = res
          return 0
        lax.fori_loop(0, C // 16, cb_body, 0)
      pltpu.sync_copy(out_v, out_hbm.at[pl.ds(base, G)])

    start(0, 0)

    def pair(t2, carry):
      t = 2 * t2
      start(t + 1, 1)
      finish(t, 0)

      @pl.when(t2 < steps // 2 - 1)
      def _():
        start(t + 2, 0)

      finish(t + 1, 1)
      return carry

    lax.fori_loop(0, steps // 2, pair, 0)

  return kern(Qf, idxf, off)


def _head(feats, Wd0, bd0, Wd1, bd1):
  B, S, C = feats.shape
  D0 = Wd0.shape[1]
  D1 = Wd1.shape[1]

  def kern(f_ref, w0_ref, b0_ref, w1_ref, b1_ref, out_ref):
    f = f_ref[0]
    fmax = jnp.max(f, axis=0, keepdims=True)
    fmean = jnp.mean(f, axis=0, keepdims=True)
    x = jnp.concatenate([fmax, fmean], axis=1)
    h = jnp.dot(x, w0_ref[...], preferred_element_type=jnp.float32)
    h = jnp.maximum(h + b0_ref[0:1, :], 0.0)
    y = jnp.dot(h, w1_ref[...], preferred_element_type=jnp.float32)
    out_ref[0] = y + b1_ref[0:1, :]

  out = pl.pallas_call(
      kern,
      grid=(B,),
      in_specs=[
          pl.BlockSpec((1, S, C), lambda b: (b, 0, 0)),
          pl.BlockSpec((2 * C, D0), lambda b: (0, 0)),
          pl.BlockSpec((1, D0), lambda b: (0, 0)),
          pl.BlockSpec((D0, D1), lambda b: (0, 0)),
          pl.BlockSpec((1, D1), lambda b: (0, 0)),
      ],
      out_specs=pl.BlockSpec((1, 1, D1), lambda b: (b, 0, 0)),
      out_shape=jax.ShapeDtypeStruct((B, 1, D1), jnp.float32),
  )(feats, Wd0, bd0, Wd1, bd1)
  return out.reshape(B, D1)


def kernel(xyz, W_proj, b_proj, W_sa0, b_sa0, W_sa1, b_sa1, W_sa2, b_sa2,
           W_d0, b_d0, W_d1, b_d1):
  B, N, _ = xyz.shape
  fps = (1024, 256, 64)

  Wf0, Wx0 = W_sa0[:-3], W_sa0[-3:]
  Wf1, Wx1 = W_sa1[:-3], W_sa1[-3:]
  Wf2, Wx2 = W_sa2[:-3], W_sa2[-3:]
  A0 = W_proj @ Wf0 + Wx0
  c0 = (b_proj @ Wf0)[None, :]

  pts0 = xyz
  c1 = pts0[:, :: N // fps[0]]
  c2 = c1[:, :: fps[0] // fps[1]]
  c3 = c2[:, :: fps[1] // fps[2]]

  C0 = Wf0.shape[1]
  Q0 = _q0(xyz, A0, c0)
  idx0, off0 = _knn_offsets(pts0, c1, Wx0, b_sa0[None, :], 256)
  f1 = _gather_max(Q0.reshape(B * N, C0), idx0.transpose(0, 2, 1).reshape(-1),
                   off0.reshape(B * fps[0], C0), C0)
  f1 = f1.reshape(B, fps[0], C0)

  C1 = Wf1.shape[1]
  Q1 = _q_layer(f1, c1, Wf1, Wx1)
  idx1, off1 = _knn_offsets(c1, c2, Wx1, b_sa1[None, :], 256)
  f2 = _gather_max(Q1.reshape(B * fps[0], C1), idx1.transpose(0, 2, 1).reshape(-1),
                   off1.reshape(B * fps[1], C1), C1)
  f2 = f2.reshape(B, fps[1], C1)

  C2 = Wf2.shape[1]
  Q2 = _q_layer(f2, c2, Wf2, Wx2)
  idx2, off2 = _knn_offsets(c2, c3, Wx2, b_sa2[None, :], 64)
  f3 = _gather_max(Q2.reshape(B * fps[1], C2), idx2.transpose(0, 2, 1).reshape(-1),
                   off2.reshape(B * fps[2], C2), C2)
  f3 = f3.reshape(B, fps[2], C2)

  return _head(f3, W_d0, b_d0[None, :], W_d1, b_d1[None, :])
